# baseline (speedup 1.0000x reference)
import functools

import jax
import jax.numpy as jnp
from jax import lax
from jax.experimental import pallas as pl
from jax.experimental.pallas import tpu as pltpu
from jax.experimental.pallas import tpu_sc as plsc

VOCAB = 1000
D = 64
BATCH = 4096
SEQ = 200
N = BATCH * SEQ

NC = 2
NS = 16
NW = NC * NS
BPW = BATCH // NW
L = 16
KB = BPW // L

_mesh = plsc.VectorSubcoreMesh(core_axis_name="c", subcore_axis_name="s")


@functools.partial(
    pl.kernel,
    out_type=jax.ShapeDtypeStruct((SEQ * D, BATCH), jnp.float32),
    mesh=_mesh,
    scratch_types=[
        pltpu.VMEM((VOCAB * D,), jnp.float32),
        pltpu.VMEM((BPW * SEQ,), jnp.int32),
        pltpu.VMEM((2, D, BPW), jnp.float32),
        pltpu.SemaphoreType.DMA,
        pltpu.SemaphoreType.DMA,
    ],
    compiler_params=pltpu.CompilerParams(needs_layout_passes=False),
)
def _embed_sc(table_hbm, idx_hbm, out_hbm, table_v, idx_v, buf_v, o0, o1):
    wid = lax.axis_index("s") * NC + lax.axis_index("c")
    b0 = wid * BPW
    osem = (o0, o1)

    pltpu.sync_copy(idx_hbm.at[pl.ds(b0 * SEQ, BPW * SEQ)], idx_v)
    pltpu.sync_copy(table_hbm, table_v)

    lane_seq = lax.iota(jnp.int32, L) * SEQ

    def drain(b):
        pltpu.make_async_copy(
            out_hbm.at[pl.ds(0, D), pl.ds(0, BPW)], buf_v.at[b], osem[b]
        ).wait()

    def assemble(s, b):
        for kb in range(KB):
            iaddr = lane_seq + (kb * L * SEQ + s)
            iv = plsc.load_gather(idx_v, [iaddr])
            ov = iv * D
            for d in range(D):
                val = plsc.load_gather(table_v, [ov + d])
                buf_v[b, d, pl.ds(kb * L, L)] = val

    def body(g, _):
        for b in range(2):
            s = 2 * g + b

            @pl.when(s >= 2)
            def _():
                drain(b)

            assemble(s, b)
            pltpu.async_copy(
                buf_v.at[b],
                out_hbm.at[pl.ds(s * D, D), pl.ds(b0, BPW)],
                osem[b],
            )
        return ()

    lax.fori_loop(0, SEQ // 2, body, ())

    for b in range(2):
        drain(b)


def kernel(table, inp):
    idx = inp.reshape(N)
    out = _embed_sc(table.reshape(VOCAB * D), idx)
    return out.reshape(SEQ, D, BATCH).transpose(2, 0, 1)


# speedup vs baseline: 1.6799x; 1.6799x over previous
import functools

import jax
import jax.numpy as jnp
from jax import lax
from jax.experimental import pallas as pl
from jax.experimental.pallas import tpu as pltpu
from jax.experimental.pallas import tpu_sc as plsc

VOCAB = 1000
D = 64
BATCH = 4096
SEQ = 200
N = BATCH * SEQ

NC = 2
NS = 16
NW = NC * NS
BPW = BATCH // NW
L = 16
KB = BPW // L

_mesh = plsc.VectorSubcoreMesh(core_axis_name="c", subcore_axis_name="s")


@functools.partial(
    pl.kernel,
    out_type=jax.ShapeDtypeStruct((SEQ * D, BATCH), jnp.float32),
    mesh=_mesh,
    scratch_types=[
        pltpu.VMEM((VOCAB * D,), jnp.float32),
        pltpu.VMEM((BPW * SEQ,), jnp.int32),
        pltpu.VMEM((2, D, BPW), jnp.float32),
        pltpu.SemaphoreType.DMA,
        pltpu.SemaphoreType.DMA,
    ],
    compiler_params=pltpu.CompilerParams(needs_layout_passes=False),
)
def _embed_sc(table_hbm, idx_hbm, out_hbm, table_v, idx_v, buf_v, o0, o1):
    wid = lax.axis_index("s") * NC + lax.axis_index("c")
    b0 = wid * BPW
    osem = (o0, o1)

    pltpu.sync_copy(idx_hbm.at[pl.ds(b0 * SEQ, BPW * SEQ)], idx_v)
    pltpu.sync_copy(table_hbm, table_v)

    lane_seq = lax.iota(jnp.int32, L) * SEQ

    def drain(b):
        pltpu.make_async_copy(
            out_hbm.at[pl.ds(0, D), pl.ds(0, BPW)], buf_v.at[b], osem[b]
        ).wait()

    GV = 8

    def assemble(s, b):
        for kb in range(KB):
            iaddr = lane_seq + (kb * L * SEQ + s)
            iv = plsc.load_gather(idx_v, [iaddr])
            ov = iv * D
            for dg in range(D // GV):
                vals = [
                    plsc.load_gather(table_v, [ov + (dg * GV + k)])
                    for k in range(GV)
                ]
                for k in range(GV):
                    buf_v[b, dg * GV + k, pl.ds(kb * L, L)] = vals[k]

    def body(g, _):
        for b in range(2):
            s = 2 * g + b

            @pl.when(s >= 2)
            def _():
                drain(b)

            assemble(s, b)
            pltpu.async_copy(
                buf_v.at[b],
                out_hbm.at[pl.ds(s * D, D), pl.ds(b0, BPW)],
                osem[b],
            )
        return ()

    lax.fori_loop(0, SEQ // 2, body, ())

    for b in range(2):
        drain(b)


def kernel(table, inp):
    idx = inp.reshape(N)
    out = _embed_sc(table.reshape(VOCAB * D), idx)
    return out.reshape(SEQ, D, BATCH).transpose(2, 0, 1)


# speedup vs baseline: 1.7180x; 1.0227x over previous
import functools

import jax
import jax.numpy as jnp
from jax import lax
from jax.experimental import pallas as pl
from jax.experimental.pallas import tpu as pltpu
from jax.experimental.pallas import tpu_sc as plsc

VOCAB = 1000
D = 64
BATCH = 4096
SEQ = 200
N = BATCH * SEQ

NC = 2
NS = 16
NBG = 16
BPT = BATCH // NBG
NSH = 2
SPT = SEQ // NSH
L = 16
KB = BPT // L
GV = 8

_mesh = plsc.VectorSubcoreMesh(core_axis_name="c", subcore_axis_name="s")


@functools.partial(
    pl.kernel,
    out_type=jax.ShapeDtypeStruct((SEQ * D, BATCH), jnp.float32),
    mesh=_mesh,
    scratch_types=[
        pltpu.VMEM((VOCAB * D,), jnp.float32),
        pltpu.VMEM((2, BPT), jnp.int32),
        pltpu.VMEM((2, D, BPT), jnp.float32),
        pltpu.SemaphoreType.DMA,
        pltpu.SemaphoreType.DMA,
        pltpu.SemaphoreType.DMA,
        pltpu.SemaphoreType.DMA,
    ],
    compiler_params=pltpu.CompilerParams(needs_layout_passes=False),
)
def _embed_sc(table_hbm, idxt_hbm, out_hbm, table_v, idx_v, buf_v, o0, o1, i0, i1):
    wid = lax.axis_index("s") * NC + lax.axis_index("c")
    b0 = (wid % NBG) * BPT
    s0 = (wid // NBG) * SPT
    osem = (o0, o1)
    isem = (i0, i1)

    pltpu.sync_copy(table_hbm, table_v)

    def fetch_idx(s, b):
        pltpu.async_copy(
            idxt_hbm.at[pl.ds(s * BATCH + b0, BPT)], idx_v.at[b], isem[b]
        )

    def drain_out(b):
        pltpu.make_async_copy(
            out_hbm.at[pl.ds(0, D), pl.ds(0, BPT)], buf_v.at[b], osem[b]
        ).wait()

    def drain_idx(b):
        pltpu.make_async_copy(
            idxt_hbm.at[pl.ds(0, BPT)], idx_v.at[b], isem[b]
        ).wait()

    def assemble(b):
        for kb in range(KB):
            iv = idx_v[b, pl.ds(kb * L, L)]
            ov = iv * D
            for dg in range(D // GV):
                vals = [
                    plsc.load_gather(table_v, [ov + (dg * GV + k)])
                    for k in range(GV)
                ]
                for k in range(GV):
                    buf_v[b, dg * GV + k, pl.ds(kb * L, L)] = vals[k]

    fetch_idx(s0, 0)
    fetch_idx(s0 + 1, 1)

    def body(g, _):
        for b in range(2):
            s = s0 + 2 * g + b
            drain_idx(b)

            @pl.when(g >= 1)
            def _():
                drain_out(b)

            assemble(b)
            pltpu.async_copy(
                buf_v.at[b],
                out_hbm.at[pl.ds(s * D, D), pl.ds(b0, BPT)],
                osem[b],
            )

            @pl.when(2 * g + b + 2 < SPT)
            def _():
                fetch_idx(s + 2, b)
        return ()

    lax.fori_loop(0, SPT // 2, body, ())

    for b in range(2):
        drain_out(b)


def kernel(table, inp):
    idx_t = inp.T.reshape(N)
    out = _embed_sc(table.reshape(VOCAB * D), idx_t)
    return out.reshape(SEQ, D, BATCH).transpose(2, 0, 1)


# speedup vs baseline: 3.6477x; 2.1232x over previous
import functools

import jax
import jax.numpy as jnp
from jax import lax
from jax.experimental import pallas as pl
from jax.experimental.pallas import tpu as pltpu
from jax.experimental.pallas import tpu_sc as plsc

VOCAB = 1000
D = 64
BATCH = 4096
SEQ = 200
N = BATCH * SEQ

NC = 2
NS = 16
NBG = 16
BPT = BATCH // NBG
NSH = 2
SPT = SEQ // NSH
L = 16
KB = BPT // L
GV = 8
DSTRIDE = D + 1

_mesh = plsc.VectorSubcoreMesh(core_axis_name="c", subcore_axis_name="s")


@functools.partial(
    pl.kernel,
    out_type=jax.ShapeDtypeStruct((SEQ * D, BATCH), jnp.float32),
    mesh=_mesh,
    scratch_types=[
        pltpu.VMEM((VOCAB * DSTRIDE,), jnp.float32),
        pltpu.VMEM((2, BPT), jnp.int32),
        pltpu.VMEM((2, D, BPT), jnp.float32),
        pltpu.SemaphoreType.DMA,
        pltpu.SemaphoreType.DMA,
        pltpu.SemaphoreType.DMA,
        pltpu.SemaphoreType.DMA,
    ],
    compiler_params=pltpu.CompilerParams(needs_layout_passes=False),
)
def _embed_sc(table_hbm, idxt_hbm, out_hbm, table_v, idx_v, buf_v, o0, o1, i0, i1):
    wid = lax.axis_index("s") * NC + lax.axis_index("c")
    b0 = (wid % NBG) * BPT
    s0 = (wid // NBG) * SPT
    osem = (o0, o1)
    isem = (i0, i1)

    pltpu.sync_copy(table_hbm, table_v)

    def fetch_idx(s, b):
        pltpu.async_copy(
            idxt_hbm.at[pl.ds(s * BATCH + b0, BPT)], idx_v.at[b], isem[b]
        )

    def drain_out(b):
        pltpu.make_async_copy(
            out_hbm.at[pl.ds(0, D), pl.ds(0, BPT)], buf_v.at[b], osem[b]
        ).wait()

    def drain_idx(b):
        pltpu.make_async_copy(
            idxt_hbm.at[pl.ds(0, BPT)], idx_v.at[b], isem[b]
        ).wait()

    def assemble(b):
        for kb in range(KB):
            iv = idx_v[b, pl.ds(kb * L, L)]
            ov = iv * DSTRIDE
            for dg in range(D // GV):
                vals = [
                    plsc.load_gather(table_v, [ov + (dg * GV + k)])
                    for k in range(GV)
                ]
                for k in range(GV):
                    buf_v[b, dg * GV + k, pl.ds(kb * L, L)] = vals[k]

    fetch_idx(s0, 0)
    fetch_idx(s0 + 1, 1)

    def body(g, _):
        for b in range(2):
            s = s0 + 2 * g + b
            drain_idx(b)

            @pl.when(g >= 1)
            def _():
                drain_out(b)

            assemble(b)
            pltpu.async_copy(
                buf_v.at[b],
                out_hbm.at[pl.ds(s * D, D), pl.ds(b0, BPT)],
                osem[b],
            )

            @pl.when(2 * g + b + 2 < SPT)
            def _():
                fetch_idx(s + 2, b)
        return ()

    lax.fori_loop(0, SPT // 2, body, ())

    for b in range(2):
        drain_out(b)


def kernel(table, inp):
    idx_t = inp.T.reshape(N)
    tpad = jnp.pad(table, ((0, 0), (0, DSTRIDE - D))).reshape(VOCAB * DSTRIDE)
    out = _embed_sc(tpad, idx_t)
    return out.reshape(SEQ, D, BATCH).transpose(2, 0, 1)


# speedup vs baseline: 3.6790x; 1.0086x over previous
import functools

import jax
import jax.numpy as jnp
from jax import lax
from jax.experimental import pallas as pl
from jax.experimental.pallas import tpu as pltpu
from jax.experimental.pallas import tpu_sc as plsc

VOCAB = 1000
D = 64
BATCH = 4096
SEQ = 200
N = BATCH * SEQ

NC = 2
NS = 16
NBG = 16
BPT = BATCH // NBG
NSH = 2
SPT = SEQ // NSH
L = 16
KB = BPT // L
GV = 8
DSTRIDE = D + 1

_mesh = plsc.VectorSubcoreMesh(core_axis_name="c", subcore_axis_name="s")


@functools.partial(
    pl.kernel,
    out_type=jax.ShapeDtypeStruct((SEQ * D, BATCH), jnp.float32),
    mesh=_mesh,
    scratch_types=[
        pltpu.VMEM((VOCAB * DSTRIDE,), jnp.float32),
        pltpu.VMEM((2, BPT), jnp.int32),
        pltpu.VMEM((2, D, BPT), jnp.float32),
        pltpu.SemaphoreType.DMA,
        pltpu.SemaphoreType.DMA,
        pltpu.SemaphoreType.DMA,
        pltpu.SemaphoreType.DMA,
    ],
    compiler_params=pltpu.CompilerParams(needs_layout_passes=False),
)
def _embed_sc(table_hbm, idxt_hbm, out_hbm, table_v, idx_v, buf_v, o0, o1, i0, i1):
    wid = lax.axis_index("s") * NC + lax.axis_index("c")
    b0 = (wid % NBG) * BPT
    s0 = (wid // NBG) * SPT
    osem = (o0, o1)
    isem = (i0, i1)

    pltpu.sync_copy(table_hbm, table_v)

    def fetch_idx(s, b):
        pltpu.async_copy(
            idxt_hbm.at[pl.ds(s * BATCH + b0, BPT)], idx_v.at[b], isem[b]
        )

    def drain_out(b):
        pltpu.make_async_copy(
            out_hbm.at[pl.ds(0, D), pl.ds(0, BPT)], buf_v.at[b], osem[b]
        ).wait()

    def drain_idx(b):
        pltpu.make_async_copy(
            idxt_hbm.at[pl.ds(0, BPT)], idx_v.at[b], isem[b]
        ).wait()

    def assemble(b):
        for kb in range(KB):
            iv = idx_v[b, pl.ds(kb * L, L)]
            ov = iv * DSTRIDE
            prev = None
            for dg in range(D // GV):
                vals = [
                    plsc.load_gather(table_v, [ov + (dg * GV + k)])
                    for k in range(GV)
                ]
                if prev is not None:
                    for k in range(GV):
                        buf_v[b, (dg - 1) * GV + k, pl.ds(kb * L, L)] = prev[k]
                prev = vals
            for k in range(GV):
                buf_v[b, D - GV + k, pl.ds(kb * L, L)] = prev[k]

    fetch_idx(s0, 0)
    fetch_idx(s0 + 1, 1)

    def body(g, _):
        for b in range(2):
            s = s0 + 2 * g + b
            drain_idx(b)

            @pl.when(g >= 1)
            def _():
                drain_out(b)

            assemble(b)
            pltpu.async_copy(
                buf_v.at[b],
                out_hbm.at[pl.ds(s * D, D), pl.ds(b0, BPT)],
                osem[b],
            )

            @pl.when(2 * g + b + 2 < SPT)
            def _():
                fetch_idx(s + 2, b)
        return ()

    lax.fori_loop(0, SPT // 2, body, ())

    for b in range(2):
        drain_out(b)


def kernel(table, inp):
    idx_t = inp.T.reshape(N)
    tpad = jnp.pad(table, ((0, 0), (0, DSTRIDE - D))).reshape(VOCAB * DSTRIDE)
    out = _embed_sc(tpad, idx_t)
    return out.reshape(SEQ, D, BATCH).transpose(2, 0, 1)
